# baseline (speedup 1.0000x reference)
import functools

import jax
import jax.numpy as jnp
from jax import lax
from jax.experimental import pallas as pl
from jax.experimental.pallas import tpu as pltpu
from jax.experimental.pallas import tpu_sc as plsc

N = 10000
NP = 10240
E = 320000
D_IN = 128
D_HID = 128
NC = 2
NS = 16
NW = NC * NS
B = 128
K = 84
EP = NW * K * B
RP = NP // NS
R = 256
GRID = NP // R

@functools.cache
def _mesh():
    return plsc.VectorSubcoreMesh(
        core_axis_name="c", subcore_axis_name="s",
        num_cores=NC, num_subcores=NS)


def _deg_body(ones_h, dst_h, zer_h, out_h, dst_v, ones_v, acc_sh, sem):
    cid = lax.axis_index("c")
    sid = lax.axis_index("s")
    wid = cid * NS + sid
    pltpu.sync_copy(dst_h.at[wid], dst_v)
    pltpu.sync_copy(ones_h, ones_v)
    pltpu.sync_copy(zer_h.at[pl.ds(sid * RP, RP)],
                    acc_sh.at[pl.ds(sid * RP, RP)])
    plsc.subcore_barrier()

    def step(gi, c):
        for b in range(6):
            pltpu.async_copy(ones_v, acc_sh.at[dst_v.at[gi * 6 + b]], sem,
                             add=True)
        for b in range(6):
            pltpu.make_async_copy(ones_v, acc_sh.at[dst_v.at[gi * 6 + b]],
                                  sem).wait()
        return c

    lax.fori_loop(0, K // 6, step, 0)
    plsc.subcore_barrier()
    pltpu.sync_copy(acc_sh.at[pl.ds(sid * RP, RP)],
                    out_h.at[cid].at[pl.ds(sid * RP, RP)])


@functools.cache
def _deg_kernel():
    return pl.kernel(
        _deg_body,
        out_type=jax.ShapeDtypeStruct((NC, NP, 128), jnp.float32),
        mesh=_mesh(),
        scratch_types=[
            pltpu.VMEM((K, B), jnp.int32),
            pltpu.VMEM((B, 128), jnp.float32),
            pltpu.VMEM_SHARED((NP, 128), jnp.float32),
            pltpu.SemaphoreType.DMA,
        ],
    )


def _make_agg(D):
    M = K // 2

    def body(g_h, src_h, dst_h, zer_h, out_h, dst_v, ring_v, buf0, buf1,
             acc_sh, sg0, sg1, sr):
        cid = lax.axis_index("c")
        sid = lax.axis_index("s")
        wid = cid * NS + sid
        pltpu.sync_copy(dst_h.at[wid], dst_v)
        pltpu.sync_copy(zer_h.at[pl.ds(sid * RP, RP)],
                        acc_sh.at[pl.ds(sid * RP, RP)])

        srcrows = src_h.at[wid]

        def ring_fetch(r):
            pltpu.async_copy(srcrows.at[r], ring_v.at[r % 4], sr)

        def ring_wait():
            pltpu.make_async_copy(srcrows.at[0], ring_v.at[0], sr).wait()

        def gather_start(j, buf, sem):
            pltpu.async_copy(g_h.at[ring_v.at[j % 4]], buf, sem)

        def gather_wait(j, buf, sem):
            pltpu.make_async_copy(g_h.at[ring_v.at[j % 4]], buf, sem).wait()

        plsc.subcore_barrier()
        for r in range(4):
            ring_fetch(r)
        ring_wait()
        gather_start(0, buf0, sg0)

        def pair(m, c):
            j0 = 2 * m
            j1 = j0 + 1
            gather_wait(j0, buf0, sg0)
            ring_wait()
            gather_start(j1, buf1, sg1)

            @pl.when(j0 + 4 < K)
            def _():
                ring_fetch(j0 + 4)

            pltpu.sync_copy(buf0, acc_sh.at[dst_v.at[j0]], add=True)
            gather_wait(j1, buf1, sg1)

            @pl.when(j0 + 2 < K)
            def _():
                ring_wait()
                gather_start(j0 + 2, buf0, sg0)

            @pl.when(j1 + 4 < K)
            def _():
                ring_fetch(j1 + 4)

            pltpu.sync_copy(buf1, acc_sh.at[dst_v.at[j1]], add=True)
            return c

        lax.fori_loop(0, M, pair, 0)
        plsc.subcore_barrier()
        pltpu.sync_copy(acc_sh.at[pl.ds(sid * RP, RP)],
                        out_h.at[cid].at[pl.ds(sid * RP, RP)])

    return pl.kernel(
        body,
        out_type=jax.ShapeDtypeStruct((NC, NP, D), jnp.float32),
        mesh=_mesh(),
        scratch_types=[
            pltpu.VMEM((K, B), jnp.int32),
            pltpu.VMEM((4, B), jnp.int32),
            pltpu.VMEM((B, D), jnp.float32),
            pltpu.VMEM((B, D), jnp.float32),
            pltpu.VMEM_SHARED((NP, D), jnp.float32),
            pltpu.SemaphoreType.DMA,
            pltpu.SemaphoreType.DMA,
            pltpu.SemaphoreType.DMA,
        ],
    )


_agg128 = functools.cache(lambda: _make_agg(D_HID))


def _tc1_body(x_ref, w_ref, p0_ref, p1_ref, g_ref, dinv_ref):
    deg = p0_ref[:, :1] + p1_ref[:, :1] + 1.0
    dinv = lax.rsqrt(deg)
    h = jnp.dot(x_ref[...], w_ref[...], preferred_element_type=jnp.float32)
    g_ref[...] = dinv * h
    dinv_ref[...] = jnp.broadcast_to(dinv, (dinv.shape[0], 16))


def _tc2_body(a0_ref, a1_ref, g_ref, dinv_ref, b1_ref, ghr_ref):
    dinv = dinv_ref[...]
    s = dinv[:, :1] * (a0_ref[...] + a1_ref[...] + g_ref[...]) + b1_ref[...]
    hr = jnp.maximum(s, 0.0)
    ghr_ref[...] = dinv[:, :1] * hr


def _tc3_body(c0_ref, c1_ref, ghr_ref, dinv_ref, w2_ref, b2_ref, out_ref):
    dinv = dinv_ref[...]
    t = c0_ref[...] + c1_ref[...] + ghr_ref[...]
    h2 = jnp.dot(t, w2_ref[...], preferred_element_type=jnp.float32)
    o = dinv[:, :1] * h2 + b2_ref[...]
    o0 = o[:, 0:1]
    o1 = o[:, 1:2]
    m = jnp.maximum(o0, o1)
    e0 = jnp.exp(o0 - m)
    e1 = jnp.exp(o1 - m)
    z = e0 + e1
    out_ref[...] = jnp.concatenate(
        [e0 / z, e1 / z, jnp.zeros((o.shape[0], 14), jnp.float32)], axis=1)


def _row_spec(d):
    return pl.BlockSpec((R, d), lambda i: (i, 0))


def _full_spec(s0, s1):
    return pl.BlockSpec((s0, s1), lambda i: (0, 0))


def _tc1(x_pad, W1, p0, p1):
    return pl.pallas_call(
        _tc1_body,
        grid=(GRID,),
        in_specs=[_row_spec(D_IN), _full_spec(D_IN, D_HID),
                  _row_spec(128), _row_spec(128)],
        out_specs=[_row_spec(D_HID), _row_spec(16)],
        out_shape=[jax.ShapeDtypeStruct((NP, D_HID), jnp.float32),
                   jax.ShapeDtypeStruct((NP, 16), jnp.float32)],
    )(x_pad, W1, p0, p1)


def _tc2(a0, a1, g, dinv16, b1r):
    return pl.pallas_call(
        _tc2_body,
        grid=(GRID,),
        in_specs=[_row_spec(D_HID), _row_spec(D_HID), _row_spec(D_HID),
                  _row_spec(16), _full_spec(1, D_HID)],
        out_specs=[_row_spec(D_HID)],
        out_shape=[jax.ShapeDtypeStruct((NP, D_HID), jnp.float32)],
    )(a0, a1, g, dinv16, b1r)[0]


def _tc3(c0, c1, ghr, dinv16, W2p, b2r):
    return pl.pallas_call(
        _tc3_body,
        grid=(GRID,),
        in_specs=[_row_spec(D_HID), _row_spec(D_HID), _row_spec(D_HID),
                  _row_spec(16), _full_spec(D_HID, 16), _full_spec(1, 16)],
        out_specs=[_row_spec(16)],
        out_shape=[jax.ShapeDtypeStruct((NP, 16), jnp.float32)],
    )(c0, c1, ghr, dinv16, W2p, b2r)[0]


def kernel(x, edge_index, W1, b1, W2, b2):
    ei = edge_index.astype(jnp.int32)
    src, dst = ei[0], ei[1]

    pad_val = N + (jnp.arange(EP - E, dtype=jnp.int32) % (NP - N))
    src_ids = jnp.concatenate([src, pad_val]).reshape(NW, K, B)
    dst_ids = jnp.concatenate([dst, pad_val]).reshape(NW, K, B)

    x_pad = jnp.pad(x, ((0, NP - N), (0, 0)))
    b1r = b1.reshape(1, D_HID)
    W2p = jnp.pad(W2, ((0, 0), (0, 16 - W2.shape[1])))
    b2r = jnp.pad(b2, (0, 16 - b2.shape[0])).reshape(1, 16)

    z128 = jnp.zeros((NP, D_HID), jnp.float32)
    ones128 = jnp.ones((B, 128), jnp.float32)

    dparts = _deg_kernel()(ones128, dst_ids, z128)
    g, dinv16 = _tc1(x_pad, W1, dparts[0], dparts[1])

    aparts = _agg128()(g, src_ids, dst_ids, z128)
    ghr = _tc2(aparts[0], aparts[1], g, dinv16, b1r)

    bparts = _agg128()(ghr, src_ids, dst_ids, z128)
    out16 = _tc3(bparts[0], bparts[1], ghr, dinv16, W2p, b2r)
    return out16[:N, :2]


# speedup vs baseline: 1.0382x; 1.0382x over previous
import functools

import jax
import jax.numpy as jnp
from jax import lax
from jax.experimental import pallas as pl
from jax.experimental.pallas import tpu as pltpu
from jax.experimental.pallas import tpu_sc as plsc

N = 10000
NP = 10240
E = 320000
D_IN = 128
D_HID = 128
NC = 2
NS = 16
NW = NC * NS
B = 128
K = 80
EP = NW * K * B
RP = NP // NS
R = 256
GRID = NP // R

@functools.cache
def _mesh():
    return plsc.VectorSubcoreMesh(
        core_axis_name="c", subcore_axis_name="s",
        num_cores=NC, num_subcores=NS)


def _deg_body(ones_h, dst_h, zer_h, out_h, dst_v, ones_v, acc_sh, sem):
    cid = lax.axis_index("c")
    sid = lax.axis_index("s")
    wid = cid * NS + sid
    pltpu.sync_copy(dst_h.at[wid], dst_v)
    pltpu.sync_copy(ones_h, ones_v)
    pltpu.sync_copy(zer_h.at[pl.ds(sid * RP, RP)],
                    acc_sh.at[pl.ds(sid * RP, RP)])
    plsc.subcore_barrier()

    def step(gi, c):
        for b in range(4):
            pltpu.async_copy(ones_v, acc_sh.at[dst_v.at[gi * 4 + b]], sem,
                             add=True)
        for b in range(4):
            pltpu.make_async_copy(ones_v, acc_sh.at[dst_v.at[gi * 4 + b]],
                                  sem).wait()
        return c

    lax.fori_loop(0, K // 4, step, 0)
    plsc.subcore_barrier()
    pltpu.sync_copy(acc_sh.at[pl.ds(sid * RP, RP)],
                    out_h.at[cid].at[pl.ds(sid * RP, RP)])


@functools.cache
def _deg_kernel():
    return pl.kernel(
        _deg_body,
        out_type=jax.ShapeDtypeStruct((NC, NP, 128), jnp.float32),
        mesh=_mesh(),
        scratch_types=[
            pltpu.VMEM((K, B), jnp.int32),
            pltpu.VMEM((B, 128), jnp.float32),
            pltpu.VMEM_SHARED((NP, 128), jnp.float32),
            pltpu.SemaphoreType.DMA,
        ],
    )


def _make_agg(D):
    M = K // 2

    def body(g_h, src_h, dst_h, zer_h, out_h, dst_v, ring_v, buf0, buf1,
             acc_sh, sg0, sg1, sr):
        cid = lax.axis_index("c")
        sid = lax.axis_index("s")
        wid = cid * NS + sid
        pltpu.sync_copy(dst_h.at[wid], dst_v)
        pltpu.sync_copy(zer_h.at[pl.ds(sid * RP, RP)],
                        acc_sh.at[pl.ds(sid * RP, RP)])

        srcrows = src_h.at[wid]

        def ring_fetch(r):
            pltpu.async_copy(srcrows.at[r], ring_v.at[r % 4], sr)

        def ring_wait():
            pltpu.make_async_copy(srcrows.at[0], ring_v.at[0], sr).wait()

        def gather_start(j, buf, sem):
            pltpu.async_copy(g_h.at[ring_v.at[j % 4]], buf, sem)

        def gather_wait(j, buf, sem):
            pltpu.make_async_copy(g_h.at[ring_v.at[j % 4]], buf, sem).wait()

        plsc.subcore_barrier()
        for r in range(4):
            ring_fetch(r)
        ring_wait()
        gather_start(0, buf0, sg0)

        def pair(m, c):
            j0 = 2 * m
            j1 = j0 + 1
            gather_wait(j0, buf0, sg0)
            ring_wait()
            gather_start(j1, buf1, sg1)

            @pl.when(j0 + 4 < K)
            def _():
                ring_fetch(j0 + 4)

            pltpu.sync_copy(buf0, acc_sh.at[dst_v.at[j0]], add=True)
            gather_wait(j1, buf1, sg1)

            @pl.when(j0 + 2 < K)
            def _():
                ring_wait()
                gather_start(j0 + 2, buf0, sg0)

            @pl.when(j1 + 4 < K)
            def _():
                ring_fetch(j1 + 4)

            pltpu.sync_copy(buf1, acc_sh.at[dst_v.at[j1]], add=True)
            return c

        lax.fori_loop(0, M, pair, 0)
        plsc.subcore_barrier()
        pltpu.sync_copy(acc_sh.at[pl.ds(sid * RP, RP)],
                        out_h.at[cid].at[pl.ds(sid * RP, RP)])

    return pl.kernel(
        body,
        out_type=jax.ShapeDtypeStruct((NC, NP, D), jnp.float32),
        mesh=_mesh(),
        scratch_types=[
            pltpu.VMEM((K, B), jnp.int32),
            pltpu.VMEM((4, B), jnp.int32),
            pltpu.VMEM((B, D), jnp.float32),
            pltpu.VMEM((B, D), jnp.float32),
            pltpu.VMEM_SHARED((NP, D), jnp.float32),
            pltpu.SemaphoreType.DMA,
            pltpu.SemaphoreType.DMA,
            pltpu.SemaphoreType.DMA,
        ],
    )


_agg128 = functools.cache(lambda: _make_agg(D_HID))


def _tc1_body(x_ref, w_ref, p0_ref, p1_ref, g_ref, dinv_ref):
    deg = p0_ref[:, :1] + p1_ref[:, :1] + 1.0
    dinv = lax.rsqrt(deg)
    h = jnp.dot(x_ref[...], w_ref[...], preferred_element_type=jnp.float32)
    g_ref[...] = dinv * h
    dinv_ref[...] = jnp.broadcast_to(dinv, (dinv.shape[0], 16))


def _tc2_body(a0_ref, a1_ref, g_ref, dinv_ref, b1_ref, ghr_ref):
    dinv = dinv_ref[...]
    s = dinv[:, :1] * (a0_ref[...] + a1_ref[...] + g_ref[...]) + b1_ref[...]
    hr = jnp.maximum(s, 0.0)
    ghr_ref[...] = dinv[:, :1] * hr


def _tc3_body(c0_ref, c1_ref, ghr_ref, dinv_ref, w2_ref, b2_ref, out_ref):
    dinv = dinv_ref[...]
    t = c0_ref[...] + c1_ref[...] + ghr_ref[...]
    h2 = jnp.dot(t, w2_ref[...], preferred_element_type=jnp.float32)
    o = dinv[:, :1] * h2 + b2_ref[...]
    o0 = o[:, 0:1]
    o1 = o[:, 1:2]
    m = jnp.maximum(o0, o1)
    e0 = jnp.exp(o0 - m)
    e1 = jnp.exp(o1 - m)
    z = e0 + e1
    out_ref[...] = jnp.concatenate(
        [e0 / z, e1 / z, jnp.zeros((o.shape[0], 14), jnp.float32)], axis=1)


def _row_spec(d):
    return pl.BlockSpec((R, d), lambda i: (i, 0))


def _full_spec(s0, s1):
    return pl.BlockSpec((s0, s1), lambda i: (0, 0))


def _tc1(x_pad, W1, p0, p1):
    return pl.pallas_call(
        _tc1_body,
        grid=(GRID,),
        in_specs=[_row_spec(D_IN), _full_spec(D_IN, D_HID),
                  _row_spec(128), _row_spec(128)],
        out_specs=[_row_spec(D_HID), _row_spec(16)],
        out_shape=[jax.ShapeDtypeStruct((NP, D_HID), jnp.float32),
                   jax.ShapeDtypeStruct((NP, 16), jnp.float32)],
    )(x_pad, W1, p0, p1)


def _tc2(a0, a1, g, dinv16, b1r):
    return pl.pallas_call(
        _tc2_body,
        grid=(GRID,),
        in_specs=[_row_spec(D_HID), _row_spec(D_HID), _row_spec(D_HID),
                  _row_spec(16), _full_spec(1, D_HID)],
        out_specs=[_row_spec(D_HID)],
        out_shape=[jax.ShapeDtypeStruct((NP, D_HID), jnp.float32)],
    )(a0, a1, g, dinv16, b1r)[0]


def _tc3(c0, c1, ghr, dinv16, W2p, b2r):
    return pl.pallas_call(
        _tc3_body,
        grid=(GRID,),
        in_specs=[_row_spec(D_HID), _row_spec(D_HID), _row_spec(D_HID),
                  _row_spec(16), _full_spec(D_HID, 16), _full_spec(1, 16)],
        out_specs=[_row_spec(16)],
        out_shape=[jax.ShapeDtypeStruct((NP, 16), jnp.float32)],
    )(c0, c1, ghr, dinv16, W2p, b2r)[0]


def kernel(x, edge_index, W1, b1, W2, b2):
    ei = edge_index.astype(jnp.int32)
    src, dst = ei[0], ei[1]

    pad_val = N + (jnp.arange(EP - E, dtype=jnp.int32) % (NP - N))
    src_ids = jnp.concatenate([src, pad_val]).reshape(NW, K, B)
    dst_ids = jnp.concatenate([dst, pad_val]).reshape(NW, K, B)

    x_pad = jnp.pad(x, ((0, NP - N), (0, 0)))
    b1r = b1.reshape(1, D_HID)
    W2p = jnp.pad(W2, ((0, 0), (0, 16 - W2.shape[1])))
    b2r = jnp.pad(b2, (0, 16 - b2.shape[0])).reshape(1, 16)

    z128 = jnp.zeros((NP, D_HID), jnp.float32)
    ones128 = jnp.ones((B, 128), jnp.float32)

    dparts = _deg_kernel()(ones128, dst_ids, z128)
    g, dinv16 = _tc1(x_pad, W1, dparts[0], dparts[1])

    aparts = _agg128()(g, src_ids, dst_ids, z128)
    ghr = _tc2(aparts[0], aparts[1], g, dinv16, b1r)

    bparts = _agg128()(ghr, src_ids, dst_ids, z128)
    out16 = _tc3(bparts[0], bparts[1], ghr, dinv16, W2p, b2r)
    return out16[:N, :2]
